# baseline (speedup 1.0000x reference)
import functools

import jax
import jax.numpy as jnp
from jax import lax
from jax.experimental import pallas as pl
from jax.experimental.pallas import tpu as pltpu
from jax.experimental.pallas import tpu_sc as plsc

N = 10000
E = 320000
D_IN = 128
D_H = 128
D_OUT = 64

NC = 2
NS = 16
NW = NC * NS
CH = 128
NCH0 = 80
NCH1 = 80
NCH_MAX = max(NCH0, NCH1)
TOT_CH = NS * (NCH0 + NCH1)
FLAT_CH = NS * NCH0 + NS * NCH1 + (NCH_MAX - min(NCH0, NCH1))
E_PAD = TOT_CH * CH
N_PAD = 10240
RPS = N_PAD // NS
DEG_W = 128
BR = 1024
GRID = N_PAD // BR

_mesh = plsc.VectorSubcoreMesh(
    core_axis_name="c", subcore_axis_name="s", num_cores=NC, num_subcores=NS)


def _wid():
  return lax.axis_index("c") * NS + lax.axis_index("s")


def _chunk_base_count(c, s):
  base = lax.select(c == 0, s * NCH0, NS * NCH0 + s * NCH1)
  count = lax.select(c == 0, NCH0, NCH1)
  return base, count


def _deg_body(dst_hbm, ones_hbm, zeros_hbm, out_hbm, acc, ones_v, idx_v, sem):
  del sem
  c = lax.axis_index("c")
  s = lax.axis_index("s")
  base, count = _chunk_base_count(c, s)
  pltpu.sync_copy(zeros_hbm, acc.at[pl.ds(s * RPS, RPS)])
  pltpu.sync_copy(ones_hbm, ones_v)
  pltpu.sync_copy(dst_hbm.at[pl.ds(base, NCH_MAX)], idx_v)
  plsc.subcore_barrier()

  def body(j, carry):
    pltpu.sync_copy(ones_v, acc.at[idx_v.at[j]], add=True)
    return carry

  lax.fori_loop(0, count, body, 0)
  plsc.subcore_barrier()
  pltpu.sync_copy(acc.at[pl.ds(s * RPS, RPS)],
                  out_hbm.at[c].at[pl.ds(s * RPS, RPS)])


_deg_pass = pl.kernel(
    _deg_body,
    out_type=jax.ShapeDtypeStruct((NC, N_PAD, DEG_W), jnp.float32),
    mesh=_mesh,
    scratch_types=[
        pltpu.VMEM_SHARED((N_PAD, DEG_W), jnp.float32),
        pltpu.VMEM((CH, DEG_W), jnp.float32),
        pltpu.VMEM((NCH_MAX, CH), jnp.int32),
        pltpu.SemaphoreType.DMA,
    ],
)


NBUF = 2
DRING = 4


def _scatter_body(D, g_hbm, src_hbm, dst_hbm, zeros_hbm, out_hbm,
                  acc, src_v, dst_r, rows, gsems, dsems):
  c = lax.axis_index("c")
  s = lax.axis_index("s")
  base, count = _chunk_base_count(c, s)
  pltpu.sync_copy(zeros_hbm, acc.at[pl.ds(s * RPS, RPS)])
  pltpu.sync_copy(src_hbm.at[pl.ds(base, NCH_MAX)], src_v)
  for k in range(DRING - 1):
    pltpu.async_copy(dst_hbm.at[base + k], dst_r.at[k], dsems[k])
  plsc.subcore_barrier()

  for k in range(NBUF - 1):
    pltpu.async_copy(g_hbm.at[src_v.at[k]], rows[k], gsems[k])

  def body(jj, carry):
    for k in range(DRING):
      j = jj * DRING + k
      kb = k % NBUF
      kn = (k + 1) % NBUF

      @pl.when(j + 1 < count)
      def _():
        pltpu.async_copy(g_hbm.at[src_v.at[j + 1]], rows[kn], gsems[kn])

      pltpu.make_async_copy(dst_hbm.at[base + j], dst_r.at[k],
                            dsems[k]).wait()
      pltpu.make_async_copy(g_hbm.at[src_v.at[j]], rows[kb], gsems[kb]).wait()
      pltpu.sync_copy(rows[kb], acc.at[dst_r.at[k]], add=True)
      ks = (k + DRING - 1) % DRING

      @pl.when(j + DRING - 1 < count)
      def _():
        pltpu.async_copy(dst_hbm.at[base + j + DRING - 1], dst_r.at[ks],
                         dsems[ks])
    return carry

  lax.fori_loop(0, count // DRING, body, 0)
  plsc.subcore_barrier()
  pltpu.sync_copy(acc.at[pl.ds(s * RPS, RPS)],
                  out_hbm.at[c].at[pl.ds(s * RPS, RPS)])


def _make_scatter(D):
  def wrapped(D, g_hbm, src_hbm, dst_hbm, zeros_hbm, out_hbm, acc,
              src_v, dst_r, r0, r1, g0, g1, d0, d1, d2, d3):
    _scatter_body(D, g_hbm, src_hbm, dst_hbm, zeros_hbm, out_hbm, acc,
                  src_v, dst_r, [r0, r1], [g0, g1], [d0, d1, d2, d3])

  return pl.kernel(
      functools.partial(wrapped, D),
      out_type=jax.ShapeDtypeStruct((NC, N_PAD, D), jnp.float32),
      mesh=_mesh,
      scratch_types=[
          pltpu.VMEM_SHARED((N_PAD, D), jnp.float32),
          pltpu.VMEM((NCH_MAX, CH), jnp.int32),
          pltpu.VMEM((DRING, CH), jnp.int32),
      ] + [pltpu.VMEM((CH, D), jnp.float32)] * NBUF
        + [pltpu.SemaphoreType.DMA] * (NBUF + DRING),
  )


_scatter128 = _make_scatter(D_H)


def _dinv_block(deg_ref, pid):
  deg = deg_ref[...]
  total = deg[0, :, 0:1] + deg[1, :, 0:1] + 1.0
  dinv = lax.rsqrt(total)
  rows = pid * BR + lax.broadcasted_iota(jnp.int32, (BR, 1), 0)
  return jnp.where(rows < N, dinv, 0.0)


def _l1_body(x_ref, deg_ref, w_ref, g_ref):
  dinv = _dinv_block(deg_ref, pl.program_id(0))
  g_ref[...] = dinv * jnp.dot(x_ref[...], w_ref[...],
                              preferred_element_type=jnp.float32)


def _mid_body(agg_ref, gprev_ref, deg_ref, w_ref, b_ref, g_ref):
  dinv = _dinv_block(deg_ref, pl.program_id(0))
  agg = agg_ref[...]
  z = dinv * (agg[0] + agg[1] + gprev_ref[...]) + b_ref[...]
  z = jnp.maximum(z, 0.0)
  g_ref[...] = dinv * jnp.dot(z, w_ref[...],
                              preferred_element_type=jnp.float32)


def _fin_body(agg_ref, gprev_ref, deg_ref, b_ref, o_ref):
  dinv = _dinv_block(deg_ref, pl.program_id(0))
  agg = agg_ref[...]
  z = (dinv * (agg[0] + agg[1] + gprev_ref[...]))[:, :D_OUT] + b_ref[...]
  z = jnp.maximum(z, 0.0)
  m = jnp.max(z, axis=1, keepdims=True)
  lse = m + jnp.log(jnp.sum(jnp.exp(z - m), axis=1, keepdims=True))
  o_ref[...] = z - lse


def _row_spec(d):
  return pl.BlockSpec((BR, d), lambda i: (i, 0))


def _pair_spec(d):
  return pl.BlockSpec((NC, BR, d), lambda i: (0, i, 0))


def _full_spec(shape):
  return pl.BlockSpec(shape, lambda i: tuple(0 for _ in shape))


def _l1_call(x, deg, w):
  return pl.pallas_call(
      _l1_body,
      grid=(GRID,),
      in_specs=[_row_spec(D_IN), _pair_spec(DEG_W), _full_spec((D_IN, D_H))],
      out_specs=_row_spec(D_H),
      out_shape=jax.ShapeDtypeStruct((N_PAD, D_H), jnp.float32),
  )(x, deg, w)


def _mid_call(agg, gprev, deg, w, b, d_out):
  d_in = gprev.shape[1]
  return pl.pallas_call(
      _mid_body,
      grid=(GRID,),
      in_specs=[_pair_spec(d_in), _row_spec(d_in), _pair_spec(DEG_W),
                _full_spec((d_in, d_out)), _full_spec((1, d_in))],
      out_specs=_row_spec(d_out),
      out_shape=jax.ShapeDtypeStruct((N_PAD, d_out), jnp.float32),
  )(agg, gprev, deg, w, b)


def _fin_call(agg, gprev, deg, b):
  return pl.pallas_call(
      _fin_body,
      grid=(GRID,),
      in_specs=[_pair_spec(D_H), _row_spec(D_H), _pair_spec(DEG_W),
                _full_spec((1, D_OUT))],
      out_specs=_row_spec(D_OUT),
      out_shape=jax.ShapeDtypeStruct((N, D_OUT), jnp.float32),
  )(agg, gprev, deg, b)


@jax.jit
def _run(x, edge_index, W1, b1, W2, b2, W3, b3):
  src = edge_index[0]
  dst = edge_index[1]
  pad = jnp.full((FLAT_CH * CH - E,), N, dtype=jnp.int32)
  src = jnp.concatenate([src, pad])
  dst = jnp.concatenate([dst, pad])
  order = jnp.argsort(src)
  srcp = src[order].reshape(FLAT_CH, CH)
  dstp = dst[order].reshape(FLAT_CH, CH)
  xp = jnp.pad(x, ((0, N_PAD - N), (0, 0)))

  z128 = jnp.zeros((RPS, D_H), jnp.float32)
  ones_deg = jnp.ones((CH, DEG_W), jnp.float32)
  z_deg = z128
  W3p = jnp.pad(W3, ((0, 0), (0, D_H - D_OUT)))

  deg = _deg_pass(dstp, ones_deg, z_deg)
  g1 = _l1_call(xp, deg, W1)
  a1 = _scatter128(g1, srcp, dstp, z128)
  g2 = _mid_call(a1, g1, deg, W2, b1.reshape(1, D_H), D_H)
  a2 = _scatter128(g2, srcp, dstp, z128)
  g3 = _mid_call(a2, g2, deg, W3p, b2.reshape(1, D_H), D_H)
  a3 = _scatter128(g3, srcp, dstp, z128)
  return _fin_call(a3, g3, deg, b3.reshape(1, D_OUT))


def kernel(x, edge_index, W1, b1, W2, b2, W3, b3):
  return _run(x, edge_index, W1, b1, W2, b2, W3, b3)


# speedup vs baseline: 1.2825x; 1.2825x over previous
import functools

import jax
import jax.numpy as jnp
from jax import lax
from jax.experimental import pallas as pl
from jax.experimental.pallas import tpu as pltpu
from jax.experimental.pallas import tpu_sc as plsc

N = 10000
E = 320000
D_IN = 128
D_H = 128
D_OUT = 64

NC = 2
NS = 16
NW = NC * NS
CH = 128
NCH0 = 80
NCH1 = 80
NCH_MAX = max(NCH0, NCH1)
TOT_CH = NS * (NCH0 + NCH1)
FLAT_CH = NS * NCH0 + NS * NCH1 + (NCH_MAX - min(NCH0, NCH1))
E_PAD = TOT_CH * CH
N_PAD = 10240
RPS = N_PAD // NS
DEG_W = 128
BR = 1024
GRID = N_PAD // BR

_mesh = plsc.VectorSubcoreMesh(
    core_axis_name="c", subcore_axis_name="s", num_cores=NC, num_subcores=NS)


def _wid():
  return lax.axis_index("c") * NS + lax.axis_index("s")


def _chunk_base_count(c, s):
  base = lax.select(c == 0, s * NCH0, NS * NCH0 + s * NCH1)
  count = lax.select(c == 0, NCH0, NCH1)
  return base, count


def _deg_body(dst_hbm, ones_hbm, zeros_hbm, out_hbm, acc, ones_v, idx_v, sem):
  c = lax.axis_index("c")
  s = lax.axis_index("s")
  base, count = _chunk_base_count(c, s)
  pltpu.sync_copy(zeros_hbm, acc.at[pl.ds(s * RPS, RPS)])
  pltpu.sync_copy(ones_hbm, ones_v)
  pltpu.sync_copy(dst_hbm.at[pl.ds(base, NCH_MAX)], idx_v)
  plsc.subcore_barrier()

  DEG_WIN = 8

  def body(j, carry):
    pltpu.async_copy(ones_v, acc.at[idx_v.at[j]], sem, add=True)

    @pl.when(j >= DEG_WIN)
    def _():
      pltpu.make_async_copy(ones_v, acc.at[idx_v.at[j - DEG_WIN]],
                            sem).wait()
    return carry

  lax.fori_loop(0, count, body, 0)
  for k in range(DEG_WIN):
    pltpu.make_async_copy(ones_v, acc.at[idx_v.at[k]], sem).wait()
  plsc.subcore_barrier()
  pltpu.sync_copy(acc.at[pl.ds(s * RPS, RPS)],
                  out_hbm.at[c].at[pl.ds(s * RPS, RPS)])


_deg_pass = pl.kernel(
    _deg_body,
    out_type=jax.ShapeDtypeStruct((NC, N_PAD, DEG_W), jnp.float32),
    mesh=_mesh,
    scratch_types=[
        pltpu.VMEM_SHARED((N_PAD, DEG_W), jnp.float32),
        pltpu.VMEM((CH, DEG_W), jnp.float32),
        pltpu.VMEM((NCH_MAX, CH), jnp.int32),
        pltpu.SemaphoreType.DMA,
    ],
)


NBUF = 2
DRING = 4


def _scatter_body(D, g_hbm, src_hbm, dst_hbm, zeros_hbm, out_hbm,
                  acc, src_v, dst_r, rows, gsems, dsems):
  c = lax.axis_index("c")
  s = lax.axis_index("s")
  base, count = _chunk_base_count(c, s)
  pltpu.sync_copy(zeros_hbm, acc.at[pl.ds(s * RPS, RPS)])
  pltpu.sync_copy(src_hbm.at[pl.ds(base, NCH_MAX)], src_v)
  for k in range(DRING - 1):
    pltpu.async_copy(dst_hbm.at[base + k], dst_r.at[k], dsems[k])
  plsc.subcore_barrier()

  for k in range(NBUF - 1):
    pltpu.async_copy(g_hbm.at[src_v.at[k]], rows[k], gsems[k])

  def body(jj, carry):
    for k in range(DRING):
      j = jj * DRING + k
      kb = k % NBUF
      kn = (k + 1) % NBUF

      @pl.when(j + 1 < count)
      def _():
        pltpu.async_copy(g_hbm.at[src_v.at[j + 1]], rows[kn], gsems[kn])

      pltpu.make_async_copy(dst_hbm.at[base + j], dst_r.at[k],
                            dsems[k]).wait()
      pltpu.make_async_copy(g_hbm.at[src_v.at[j]], rows[kb], gsems[kb]).wait()
      pltpu.sync_copy(rows[kb], acc.at[dst_r.at[k]], add=True)
      ks = (k + DRING - 1) % DRING

      @pl.when(j + DRING - 1 < count)
      def _():
        pltpu.async_copy(dst_hbm.at[base + j + DRING - 1], dst_r.at[ks],
                         dsems[ks])
    return carry

  lax.fori_loop(0, count // DRING, body, 0)
  plsc.subcore_barrier()
  pltpu.sync_copy(acc.at[pl.ds(s * RPS, RPS)],
                  out_hbm.at[c].at[pl.ds(s * RPS, RPS)])


def _make_scatter(D):
  def wrapped(D, g_hbm, src_hbm, dst_hbm, zeros_hbm, out_hbm, acc,
              src_v, dst_r, r0, r1, g0, g1, d0, d1, d2, d3):
    _scatter_body(D, g_hbm, src_hbm, dst_hbm, zeros_hbm, out_hbm, acc,
                  src_v, dst_r, [r0, r1], [g0, g1], [d0, d1, d2, d3])

  return pl.kernel(
      functools.partial(wrapped, D),
      out_type=jax.ShapeDtypeStruct((NC, N_PAD, D), jnp.float32),
      mesh=_mesh,
      scratch_types=[
          pltpu.VMEM_SHARED((N_PAD, D), jnp.float32),
          pltpu.VMEM((NCH_MAX, CH), jnp.int32),
          pltpu.VMEM((DRING, CH), jnp.int32),
      ] + [pltpu.VMEM((CH, D), jnp.float32)] * NBUF
        + [pltpu.SemaphoreType.DMA] * (NBUF + DRING),
  )


_scatter128 = _make_scatter(D_H)


def _dinv_block(deg_ref, pid):
  deg = deg_ref[...]
  total = deg[0, :, 0:1] + deg[1, :, 0:1] + 1.0
  dinv = lax.rsqrt(total)
  rows = pid * BR + lax.broadcasted_iota(jnp.int32, (BR, 1), 0)
  return jnp.where(rows < N, dinv, 0.0)


def _l1_body(x_ref, deg_ref, w_ref, g_ref):
  dinv = _dinv_block(deg_ref, pl.program_id(0))
  g_ref[...] = dinv * jnp.dot(x_ref[...], w_ref[...],
                              preferred_element_type=jnp.float32)


def _mid_body(agg_ref, gprev_ref, deg_ref, w_ref, b_ref, g_ref):
  dinv = _dinv_block(deg_ref, pl.program_id(0))
  agg = agg_ref[...]
  z = dinv * (agg[0] + agg[1] + gprev_ref[...]) + b_ref[...]
  z = jnp.maximum(z, 0.0)
  g_ref[...] = dinv * jnp.dot(z, w_ref[...],
                              preferred_element_type=jnp.float32)


def _fin_body(agg_ref, gprev_ref, deg_ref, b_ref, o_ref):
  dinv = _dinv_block(deg_ref, pl.program_id(0))
  agg = agg_ref[...]
  z = (dinv * (agg[0] + agg[1] + gprev_ref[...]))[:, :D_OUT] + b_ref[...]
  z = jnp.maximum(z, 0.0)
  m = jnp.max(z, axis=1, keepdims=True)
  lse = m + jnp.log(jnp.sum(jnp.exp(z - m), axis=1, keepdims=True))
  o_ref[...] = z - lse


def _row_spec(d):
  return pl.BlockSpec((BR, d), lambda i: (i, 0))


def _pair_spec(d):
  return pl.BlockSpec((NC, BR, d), lambda i: (0, i, 0))


def _full_spec(shape):
  return pl.BlockSpec(shape, lambda i: tuple(0 for _ in shape))


def _l1_call(x, deg, w):
  return pl.pallas_call(
      _l1_body,
      grid=(GRID,),
      in_specs=[_row_spec(D_IN), _pair_spec(DEG_W), _full_spec((D_IN, D_H))],
      out_specs=_row_spec(D_H),
      out_shape=jax.ShapeDtypeStruct((N_PAD, D_H), jnp.float32),
  )(x, deg, w)


def _mid_call(agg, gprev, deg, w, b, d_out):
  d_in = gprev.shape[1]
  return pl.pallas_call(
      _mid_body,
      grid=(GRID,),
      in_specs=[_pair_spec(d_in), _row_spec(d_in), _pair_spec(DEG_W),
                _full_spec((d_in, d_out)), _full_spec((1, d_in))],
      out_specs=_row_spec(d_out),
      out_shape=jax.ShapeDtypeStruct((N_PAD, d_out), jnp.float32),
  )(agg, gprev, deg, w, b)


def _fin_call(agg, gprev, deg, b):
  return pl.pallas_call(
      _fin_body,
      grid=(GRID,),
      in_specs=[_pair_spec(D_H), _row_spec(D_H), _pair_spec(DEG_W),
                _full_spec((1, D_OUT))],
      out_specs=_row_spec(D_OUT),
      out_shape=jax.ShapeDtypeStruct((N, D_OUT), jnp.float32),
  )(agg, gprev, deg, b)


@jax.jit
def _run(x, edge_index, W1, b1, W2, b2, W3, b3):
  src = edge_index[0]
  dst = edge_index[1]
  pad = jnp.full((FLAT_CH * CH - E,), N, dtype=jnp.int32)
  srcp = jnp.concatenate([src, pad]).reshape(FLAT_CH, CH)
  dstp = jnp.concatenate([dst, pad]).reshape(FLAT_CH, CH)
  xp = jnp.pad(x, ((0, N_PAD - N), (0, 0)))

  z128 = jnp.zeros((RPS, D_H), jnp.float32)
  ones_deg = jnp.ones((CH, DEG_W), jnp.float32)
  z_deg = z128
  W3p = jnp.pad(W3, ((0, 0), (0, D_H - D_OUT)))

  deg = _deg_pass(dstp, ones_deg, z_deg)
  g1 = _l1_call(xp, deg, W1)
  a1 = _scatter128(g1, srcp, dstp, z128)
  g2 = _mid_call(a1, g1, deg, W2, b1.reshape(1, D_H), D_H)
  a2 = _scatter128(g2, srcp, dstp, z128)
  g3 = _mid_call(a2, g2, deg, W3p, b2.reshape(1, D_H), D_H)
  a3 = _scatter128(g3, srcp, dstp, z128)
  return _fin_call(a3, g3, deg, b3.reshape(1, D_OUT))


def kernel(x, edge_index, W1, b1, W2, b2, W3, b3):
  return _run(x, edge_index, W1, b1, W2, b2, W3, b3)


# speedup vs baseline: 1.3880x; 1.0823x over previous
import functools

import jax
import jax.numpy as jnp
from jax import lax
from jax.experimental import pallas as pl
from jax.experimental.pallas import tpu as pltpu
from jax.experimental.pallas import tpu_sc as plsc

N = 10000
E = 320000
D_IN = 128
D_H = 128
D_OUT = 64

NC = 2
NS = 16
NW = NC * NS
CH = 128
NCH = 80
E_PAD = NW * NCH * CH
N_PAD = 10240
RPS = N_PAD // NS
DEG_W = 128
BR = 1024
GRID = N_PAD // BR

_mesh = plsc.VectorSubcoreMesh(
    core_axis_name="c", subcore_axis_name="s", num_cores=NC, num_subcores=NS)


def _wid():
  return lax.axis_index("c") * NS + lax.axis_index("s")


def _deg_body(dst_hbm, ones_hbm, zeros_hbm, out_hbm, acc, ones_v, idx_v, sem):
  del sem
  c = lax.axis_index("c")
  s = lax.axis_index("s")
  wid = c * NS + s
  pltpu.sync_copy(zeros_hbm, acc.at[pl.ds(s * RPS, RPS)])
  pltpu.sync_copy(ones_hbm, ones_v)
  pltpu.sync_copy(dst_hbm.at[wid], idx_v)
  plsc.subcore_barrier()

  def body(j, carry):
    pltpu.sync_copy(ones_v, acc.at[idx_v.at[j]], add=True)
    return carry

  lax.fori_loop(0, NCH, body, 0)
  plsc.subcore_barrier()
  pltpu.sync_copy(acc.at[pl.ds(s * RPS, RPS)],
                  out_hbm.at[c].at[pl.ds(s * RPS, RPS)])


_deg_pass = pl.kernel(
    _deg_body,
    out_type=jax.ShapeDtypeStruct((NC, N_PAD, DEG_W), jnp.float32),
    mesh=_mesh,
    scratch_types=[
        pltpu.VMEM_SHARED((N_PAD, DEG_W), jnp.float32),
        pltpu.VMEM((CH, DEG_W), jnp.float32),
        pltpu.VMEM((NCH, CH), jnp.int32),
        pltpu.SemaphoreType.DMA,
    ],
)


NBUF = 2
DRING = 4


def _scatter_body(D, g_hbm, src_hbm, dst_hbm, zeros_hbm, out_hbm,
                  acc, src_v, dst_r, rows, gsems, dsems):
  c = lax.axis_index("c")
  s = lax.axis_index("s")
  wid = c * NS + s
  pltpu.sync_copy(zeros_hbm, acc.at[pl.ds(s * RPS, RPS)])
  pltpu.sync_copy(src_hbm.at[wid], src_v)
  for k in range(DRING - 1):
    pltpu.async_copy(dst_hbm.at[wid].at[k], dst_r.at[k], dsems[k])
  plsc.subcore_barrier()

  for k in range(NBUF - 1):
    pltpu.async_copy(g_hbm.at[src_v.at[k]], rows[k], gsems[k])

  def body(jj, carry):
    for k in range(DRING):
      j = jj * DRING + k
      kb = k % NBUF
      kn = (k + 1) % NBUF

      @pl.when(j + 1 < NCH)
      def _():
        pltpu.async_copy(g_hbm.at[src_v.at[j + 1]], rows[kn], gsems[kn])

      pltpu.make_async_copy(dst_hbm.at[wid].at[j], dst_r.at[k],
                            dsems[k]).wait()
      pltpu.make_async_copy(g_hbm.at[src_v.at[j]], rows[kb], gsems[kb]).wait()
      pltpu.sync_copy(rows[kb], acc.at[dst_r.at[k]], add=True)
      ks = (k + DRING - 1) % DRING

      @pl.when(j + DRING - 1 < NCH)
      def _():
        pltpu.async_copy(dst_hbm.at[wid].at[j + DRING - 1], dst_r.at[ks],
                         dsems[ks])
    return carry

  lax.fori_loop(0, NCH // DRING, body, 0)
  plsc.subcore_barrier()
  pltpu.sync_copy(acc.at[pl.ds(s * RPS, RPS)],
                  out_hbm.at[c].at[pl.ds(s * RPS, RPS)])


def _make_scatter(D):
  def wrapped(D, g_hbm, src_hbm, dst_hbm, zeros_hbm, out_hbm, acc,
              src_v, dst_r, r0, r1, g0, g1, d0, d1, d2, d3):
    _scatter_body(D, g_hbm, src_hbm, dst_hbm, zeros_hbm, out_hbm, acc,
                  src_v, dst_r, [r0, r1], [g0, g1], [d0, d1, d2, d3])

  return pl.kernel(
      functools.partial(wrapped, D),
      out_type=jax.ShapeDtypeStruct((NC, N_PAD, D), jnp.float32),
      mesh=_mesh,
      scratch_types=[
          pltpu.VMEM_SHARED((N_PAD, D), jnp.float32),
          pltpu.VMEM((NCH, CH), jnp.int32),
          pltpu.VMEM((DRING, CH), jnp.int32),
      ] + [pltpu.VMEM((CH, D), jnp.float32)] * NBUF
        + [pltpu.SemaphoreType.DMA] * (NBUF + DRING),
  )


_scatter128 = _make_scatter(D_H)


def _dinv_block(deg_ref, pid):
  deg = deg_ref[...]
  total = deg[0, :, 0:1] + deg[1, :, 0:1] + 1.0
  dinv = lax.rsqrt(total)
  rows = pid * BR + lax.broadcasted_iota(jnp.int32, (BR, 1), 0)
  return jnp.where(rows < N, dinv, 0.0)


def _l1_body(x_ref, deg_ref, w_ref, g_ref):
  dinv = _dinv_block(deg_ref, pl.program_id(0))
  g_ref[...] = dinv * jnp.dot(x_ref[...], w_ref[...],
                              preferred_element_type=jnp.float32)


def _mid_body(agg_ref, gprev_ref, deg_ref, w_ref, b_ref, g_ref):
  dinv = _dinv_block(deg_ref, pl.program_id(0))
  agg = agg_ref[...]
  z = dinv * (agg[0] + agg[1] + gprev_ref[...]) + b_ref[...]
  z = jnp.maximum(z, 0.0)
  g_ref[...] = dinv * jnp.dot(z, w_ref[...],
                              preferred_element_type=jnp.float32)


def _fin_body(agg_ref, gprev_ref, deg_ref, b_ref, o_ref):
  dinv = _dinv_block(deg_ref, pl.program_id(0))
  agg = agg_ref[...]
  z = (dinv * (agg[0] + agg[1] + gprev_ref[...]))[:, :D_OUT] + b_ref[...]
  z = jnp.maximum(z, 0.0)
  m = jnp.max(z, axis=1, keepdims=True)
  lse = m + jnp.log(jnp.sum(jnp.exp(z - m), axis=1, keepdims=True))
  o_ref[...] = z - lse


def _row_spec(d):
  return pl.BlockSpec((BR, d), lambda i: (i, 0))


def _pair_spec(d):
  return pl.BlockSpec((NC, BR, d), lambda i: (0, i, 0))


def _full_spec(shape):
  return pl.BlockSpec(shape, lambda i: tuple(0 for _ in shape))


def _l1_call(x, deg, w):
  return pl.pallas_call(
      _l1_body,
      grid=(GRID,),
      in_specs=[_row_spec(D_IN), _pair_spec(DEG_W), _full_spec((D_IN, D_H))],
      out_specs=_row_spec(D_H),
      out_shape=jax.ShapeDtypeStruct((N_PAD, D_H), jnp.float32),
  )(x, deg, w)


def _mid_call(agg, gprev, deg, w, b, d_out):
  d_in = gprev.shape[1]
  return pl.pallas_call(
      _mid_body,
      grid=(GRID,),
      in_specs=[_pair_spec(d_in), _row_spec(d_in), _pair_spec(DEG_W),
                _full_spec((d_in, d_out)), _full_spec((1, d_in))],
      out_specs=_row_spec(d_out),
      out_shape=jax.ShapeDtypeStruct((N_PAD, d_out), jnp.float32),
  )(agg, gprev, deg, w, b)


def _fin_call(agg, gprev, deg, b):
  return pl.pallas_call(
      _fin_body,
      grid=(GRID,),
      in_specs=[_pair_spec(D_H), _row_spec(D_H), _pair_spec(DEG_W),
                _full_spec((1, D_OUT))],
      out_specs=_row_spec(D_OUT),
      out_shape=jax.ShapeDtypeStruct((N, D_OUT), jnp.float32),
  )(agg, gprev, deg, b)


@jax.jit
def _run(x, edge_index, W1, b1, W2, b2, W3, b3):
  src = edge_index[0]
  dst = edge_index[1]
  pad = jnp.full((E_PAD - E,), N, dtype=jnp.int32)
  srcp = jnp.concatenate([src, pad]).reshape(NW, NCH, CH)
  dstp = jnp.concatenate([dst, pad]).reshape(NW, NCH, CH)
  xp = jnp.pad(x, ((0, N_PAD - N), (0, 0)))

  z128 = jnp.zeros((RPS, D_H), jnp.float32)
  ones_deg = jnp.ones((CH, DEG_W), jnp.float32)
  z_deg = z128
  W3p = jnp.pad(W3, ((0, 0), (0, D_H - D_OUT)))

  deg = _deg_pass(dstp, ones_deg, z_deg)
  g1 = _l1_call(xp, deg, W1)
  a1 = _scatter128(g1, srcp, dstp, z128)
  g2 = _mid_call(a1, g1, deg, W2, b1.reshape(1, D_H), D_H)
  a2 = _scatter128(g2, srcp, dstp, z128)
  g3 = _mid_call(a2, g2, deg, W3p, b2.reshape(1, D_H), D_H)
  a3 = _scatter128(g3, srcp, dstp, z128)
  return _fin_call(a3, g3, deg, b3.reshape(1, D_OUT))


def kernel(x, edge_index, W1, b1, W2, b2, W3, b3):
  return _run(x, edge_index, W1, b1, W2, b2, W3, b3)
